# baseline (speedup 1.0000x reference)
import numpy as np

import jax
import jax.numpy as jnp
from jax import lax
from jax.experimental import pallas as pl
from jax.experimental.pallas import tpu as pltpu
from jax.experimental.pallas import tpu_sc as plsc

N0, N1, N2 = 50000, 20000, 5000
D_IN, D_H = 128, 50
DP = 64
DT = 128
CNT = 50
NC, NS, L = 2, 16, 16
NW = NC * NS
CH = 128

R0 = 20480
R1 = 6144


def _ceil_div(a, b):
    return (a + b - 1) // b


def _tab_body(x_ref, w_ref, o_ref):
    acc = jnp.dot(x_ref[...], w_ref[...], preferred_element_type=jnp.float32)
    col = lax.broadcasted_iota(jnp.int32, (1, DT), 1)
    o_ref[...] = (acc + jnp.where(col % DP == CNT, 1.0, 0.0)).astype(
        jnp.bfloat16)


def _make_table(x, w_dup, block_rows):
    n, d = x.shape
    return pl.pallas_call(
        _tab_body,
        grid=(n // block_rows,),
        in_specs=[
            pl.BlockSpec((block_rows, d), lambda i: (i, 0)),
            pl.BlockSpec((d, DT), lambda i: (0, 0)),
        ],
        out_specs=pl.BlockSpec((block_rows, DT), lambda i: (i, 0)),
        out_shape=jax.ShapeDtypeStruct((n, DT), jnp.bfloat16),
    )(x, w_dup)


def _make_sc_agg(n_chunks, n_rows):
    rows_per_tile = n_rows // NS
    n_zch = rows_per_tile // CH
    mesh = plsc.VectorSubcoreMesh(
        core_axis_name="c", subcore_axis_name="s",
        num_cores=NC, num_subcores=NS)
    NB = 3
    assert n_chunks % NB == 0 and n_chunks >= 2 * NB

    def body(edge_hbm, tab_hbm, out_hbm,
             idx_s, idx_d, rows0, rows1, rows2, acc,
             g0, g1, g2, s0, s1, s2):
        rows = (rows0, rows1, rows2)
        gsem = (g0, g1, g2)
        ssem = (s0, s1, s2)
        zbuf = rows0
        c = lax.axis_index("c")
        s = lax.axis_index("s")
        w = c * NS + s

        zv = jnp.zeros((2 * L,), jnp.bfloat16)

        def zrow(i, carry):
            for k in range(DP // (2 * L)):
                zbuf[i, pl.ds(k * 2 * L, 2 * L)] = zv
            return carry
        lax.fori_loop(0, CH, zrow, 0)

        def zch(k, carry):
            pltpu.sync_copy(
                zbuf, acc.at[pl.ds(s * rows_per_tile + k * CH, CH)])
            return carry
        lax.fori_loop(0, n_zch, zch, 0)

        pltpu.sync_copy(edge_hbm.at[0, w], idx_s)
        pltpu.sync_copy(edge_hbm.at[1, w], idx_d)

        def scl(j, carry):
            for k in range(CH // L):
                sl = pl.ds(k * L, L)
                idx_s[j, sl] = idx_s[j, sl] * 2
            return carry
        lax.fori_loop(0, n_chunks, scl, 0)
        plsc.subcore_barrier()

        def fire_g(j, b):
            pltpu.async_copy(tab_hbm.at[idx_s.at[j]], rows[b], gsem[b])

        def wait_g(b):
            pltpu.make_async_copy(
                tab_hbm.at[idx_s.at[0]], rows[b], gsem[b]).wait()

        def fire_s(j, b):
            pltpu.async_copy(
                rows[b], acc.at[idx_d.at[j]], ssem[b], add=True)

        def wait_s(b):
            pltpu.make_async_copy(
                rows[b], acc.at[idx_d.at[0]], ssem[b]).wait()

        for b in range(NB):
            fire_g(b, b)

        def grp(g, carry):
            j = NB * g
            for b in range(NB):
                wait_g(b)
                fire_s(j + b, b)
            for b in range(NB):
                wait_s(b)
                fire_g(j + NB + b, b)
            return carry
        lax.fori_loop(0, n_chunks // NB - 1, grp, 0)
        j_last = n_chunks - NB
        for b in range(NB):
            wait_g(b)
            fire_s(j_last + b, b)
        for b in range(NB):
            wait_s(b)
        plsc.subcore_barrier()

        pltpu.sync_copy(
            acc.at[pl.ds(s * rows_per_tile, rows_per_tile)],
            out_hbm.at[c, pl.ds(s * rows_per_tile, rows_per_tile),
                       pl.ds(0, DP)])

    return pl.kernel(
        body,
        out_type=jax.ShapeDtypeStruct((NC, n_rows, DT), jnp.bfloat16),
        mesh=mesh,
        scratch_types=[
            pltpu.VMEM((n_chunks, CH), jnp.int32),
            pltpu.VMEM((n_chunks, CH), jnp.int32),
            pltpu.VMEM((CH, DP), jnp.bfloat16),
            pltpu.VMEM((CH, DP), jnp.bfloat16),
            pltpu.VMEM((CH, DP), jnp.bfloat16),
            pltpu.VMEM_SHARED((n_rows, DP), jnp.bfloat16),
        ] + [pltpu.SemaphoreType.DMA] * 6,
        compiler_params=pltpu.CompilerParams(use_tc_tiling_on_sc=False),
    )


def _pad_edges(edge_index, n_chunks, dummy_lo, dummy_hi, n_src):
    e_pad = NW * n_chunks * CH
    pad = e_pad - edge_index.shape[1]
    ar = np.arange(pad, dtype=np.int32)
    pad_blk = jnp.asarray(np.stack([
        ar % n_src,
        dummy_lo + ar % (dummy_hi - dummy_lo),
    ]), jnp.int32)
    return jnp.concatenate([edge_index, pad_blk], axis=1).reshape(
        2, NW, n_chunks, CH)


def _tc2_body(p_ref, x_ref, wr_ref, bl_ref, wl_ref, hl_ref, h_ref):
    col = lax.broadcasted_iota(jnp.int32, (1, DT), 1)
    psum = p_ref[0].astype(jnp.float32) + p_ref[1].astype(jnp.float32)
    sfull = jnp.where(col < DP, psum, 0.0)
    cnt = jnp.maximum(sfull[:, CNT:CNT + 1], 1.0)
    mean = jnp.where(col < CNT, sfull / cnt, 0.0)
    xw = jnp.dot(x_ref[...], wr_ref[...], preferred_element_type=jnp.float32)
    h = jnp.maximum(mean + bl_ref[...] + xw, 0.0)
    h_ref[...] = h
    hl_ref[...] = (
        jnp.dot(h, wl_ref[...], preferred_element_type=jnp.float32)
        + jnp.where(col % DP == CNT, 1.0, 0.0)).astype(jnp.bfloat16)


def _tc3_body(q_ref, h_ref, wr_ref, bl_ref, wo_ref, bo_ref, o_ref):
    col = lax.broadcasted_iota(jnp.int32, (1, DT), 1)
    qsum = q_ref[0].astype(jnp.float32) + q_ref[1].astype(jnp.float32)
    sfull = jnp.where(col < DP, qsum, 0.0)
    cnt = jnp.maximum(sfull[:, CNT:CNT + 1], 1.0)
    mean = jnp.where(col < CNT, sfull / cnt, 0.0)
    hw = jnp.dot(h_ref[...], wr_ref[...], preferred_element_type=jnp.float32)
    pre = mean + bl_ref[...] + hw
    out = jnp.dot(pre, wo_ref[...], preferred_element_type=jnp.float32)
    o_ref[...] = jnp.maximum(out + bo_ref[...], 0.0)


def kernel(x, edge_index_0, edge_index_1, edge_attr,
           Wl0, bl0, Wr0, Wl1, bl1, Wr1, W_out, b_out):
    del edge_attr
    f32 = jnp.float32

    def pad64(w):
        out = jnp.zeros((w.shape[0], DP), f32)
        return out.at[:, :w.shape[1]].set(w)

    wl0_d = jnp.concatenate([pad64(Wl0)] * 2, axis=1)
    wr0_d = jnp.concatenate([pad64(Wr0)] * 2, axis=1)
    wl1_p = jnp.zeros((DP, DP), f32).at[:D_H, :D_H].set(Wl1)
    wr1_p = jnp.zeros((DP, DP), f32).at[:D_H, :D_H].set(Wr1)
    zz = jnp.zeros((DP, DP), f32)
    wl1_bd = jnp.concatenate([
        jnp.concatenate([wl1_p, zz], axis=1),
        jnp.concatenate([zz, wl1_p], axis=1)], axis=0)
    wr1_bd = jnp.concatenate([
        jnp.concatenate([wr1_p, zz], axis=1),
        jnp.concatenate([zz, wr1_p], axis=1)], axis=0)
    wo_s = jnp.zeros((DT, 1), f32).at[:D_H, :].set(W_out)
    bl0_d = jnp.zeros((1, DT), f32).at[0, :D_H].set(bl0)
    bl1_d = jnp.zeros((1, DT), f32).at[0, :D_H].set(bl1)
    bo = b_out.reshape(1, 1)

    e0 = edge_index_0.shape[1]
    e1 = edge_index_1.shape[1]
    nch0 = 3 * _ceil_div(_ceil_div(e0, NW), 3 * CH)
    nch1 = 3 * _ceil_div(_ceil_div(e1, NW), 3 * CH)
    ei0 = _pad_edges(edge_index_0, nch0, N1, R0, N0)
    ei1 = _pad_edges(edge_index_1, nch1, N2, R1, N1)

    xt = _make_table(x, wl0_d, 5000)
    tab0 = xt.reshape(2 * N0, DP)

    p0 = _make_sc_agg(nch0, R0)(ei0, tab0)

    b2 = 4000
    hl, h = pl.pallas_call(
        _tc2_body,
        grid=(N1 // b2,),
        in_specs=[
            pl.BlockSpec((NC, b2, DT), lambda i: (0, i, 0)),
            pl.BlockSpec((b2, D_IN), lambda i: (i, 0)),
            pl.BlockSpec((D_IN, DT), lambda i: (0, 0)),
            pl.BlockSpec((1, DT), lambda i: (0, 0)),
            pl.BlockSpec((DT, DT), lambda i: (0, 0)),
        ],
        out_specs=[
            pl.BlockSpec((b2, DT), lambda i: (i, 0)),
            pl.BlockSpec((b2, DT), lambda i: (i, 0)),
        ],
        out_shape=[
            jax.ShapeDtypeStruct((N1, DT), jnp.bfloat16),
            jax.ShapeDtypeStruct((N1, DT), f32),
        ],
    )(p0, x, wr0_d, bl0_d, wl1_bd)

    tab1 = hl.reshape(2 * N1, DP)
    p1 = _make_sc_agg(nch1, R1)(ei1, tab1)

    out = pl.pallas_call(
        _tc3_body,
        grid=(1,),
        in_specs=[
            pl.BlockSpec((NC, N2, DT), lambda i: (0, 0, 0)),
            pl.BlockSpec((N2, DT), lambda i: (0, 0)),
            pl.BlockSpec((DT, DT), lambda i: (0, 0)),
            pl.BlockSpec((1, DT), lambda i: (0, 0)),
            pl.BlockSpec((DT, 1), lambda i: (0, 0)),
            pl.BlockSpec((1, 1), lambda i: (0, 0)),
        ],
        out_specs=pl.BlockSpec((N2, 1), lambda i: (0, 0)),
        out_shape=jax.ShapeDtypeStruct((N2, 1), f32),
    )(p1, h, wr1_bd, bl1_d, wo_s, bo)

    return out


# speedup vs baseline: 1.6439x; 1.6439x over previous
import numpy as np

import jax
import jax.numpy as jnp
from jax import lax
from jax.experimental import pallas as pl
from jax.experimental.pallas import tpu as pltpu
from jax.experimental.pallas import tpu_sc as plsc

N0, N1, N2 = 50000, 20000, 5000
D_IN, D_H = 128, 50
DP = 64
DT = 128
CNT = 50
NC, NS, L = 2, 16, 16
NW = NC * NS
CH = 128

R0 = 20480
R1 = 6144


def _ceil_div(a, b):
    return (a + b - 1) // b


def _tab_body(x1_ref, x2_ref, wl_ref, wr_ref, o_ref):
    acc = (jnp.dot(x1_ref[...], wl_ref[...], preferred_element_type=jnp.float32)
           + jnp.dot(x2_ref[...], wr_ref[...],
                     preferred_element_type=jnp.float32))
    col = lax.broadcasted_iota(jnp.int32, (1, DT), 1)
    o_ref[...] = acc + jnp.where(col % DP == CNT, 1.0, 0.0)


def _make_table(x, w_left, w_right, block_rows):
    n, d = x.shape
    half_blocks = (n // 2) // block_rows
    return pl.pallas_call(
        _tab_body,
        grid=(half_blocks,),
        in_specs=[
            pl.BlockSpec((block_rows, d), lambda i: (i, 0)),
            pl.BlockSpec((block_rows, d),
                         lambda i, hb=half_blocks: (i + hb, 0)),
            pl.BlockSpec((d, DT), lambda i: (0, 0)),
            pl.BlockSpec((d, DT), lambda i: (0, 0)),
        ],
        out_specs=pl.BlockSpec((block_rows, DT), lambda i: (i, 0)),
        out_shape=jax.ShapeDtypeStruct((n // 2, DT), jnp.float32),
    )(x, x, w_left, w_right)


def _make_sc_agg(n_chunks, n_rows, split_half):
    rows_per_tile = n_rows // NS
    n_zch = rows_per_tile // CH
    mesh = plsc.VectorSubcoreMesh(
        core_axis_name="c", subcore_axis_name="s",
        num_cores=NC, num_subcores=NS)
    NB = 3
    assert n_chunks % NB == 0 and n_chunks >= 2 * NB

    def body(edge_hbm, tab_hbm, out_hbm,
             idx_s, idx_d, rows0, rows1, rows2, acc,
             g0, g1, g2, s0, s1, s2):
        rows = (rows0, rows1, rows2)
        gsem = (g0, g1, g2)
        ssem = (s0, s1, s2)
        zbuf = rows0
        c = lax.axis_index("c")
        s = lax.axis_index("s")
        w = c * NS + s

        zv = jnp.zeros((L,), jnp.float32)

        def zrow(i, carry):
            for k in range(DP // L):
                zbuf[i, pl.ds(k * L, L)] = zv
            return carry
        lax.fori_loop(0, CH, zrow, 0)

        def zch(k, carry):
            pltpu.async_copy(
                zbuf, acc.at[pl.ds(s * rows_per_tile + k * CH, CH)], g0)
            return carry
        lax.fori_loop(0, n_zch, zch, 0)
        pltpu.sync_copy(edge_hbm.at[0, w], idx_s)
        pltpu.sync_copy(edge_hbm.at[1, w], idx_d)

        def zdr(k, carry):
            pltpu.make_async_copy(
                zbuf, acc.at[pl.ds(s * rows_per_tile, CH)], g0).wait()
            return carry
        lax.fori_loop(0, n_zch, zdr, 0)
        plsc.subcore_barrier()

        def fire_g(j, b):
            for k in range(CH // L):
                sl = pl.ds(k * L, L)
                v = idx_s[j, sl]
                if split_half is None:
                    idx_s[j, sl] = v * 2
                else:
                    adj = jnp.where(v >= split_half,
                                    jnp.full((L,), 2 * split_half - 1,
                                             jnp.int32),
                                    jnp.zeros((L,), jnp.int32))
                    idx_s[j, sl] = v * 2 - adj
            pltpu.async_copy(tab_hbm.at[idx_s.at[j]], rows[b], gsem[b])

        def wait_g(b):
            pltpu.make_async_copy(
                tab_hbm.at[idx_s.at[0]], rows[b], gsem[b]).wait()

        def fire_s(j, b):
            pltpu.async_copy(
                rows[b], acc.at[idx_d.at[j]], ssem[b], add=True)

        def wait_s(b):
            pltpu.make_async_copy(
                rows[b], acc.at[idx_d.at[0]], ssem[b]).wait()

        for b in range(NB):
            fire_g(b, b)

        def grp(g, carry):
            j = NB * g
            for b in range(NB):
                wait_g(b)
                fire_s(j + b, b)
            for b in range(NB):
                wait_s(b)
                fire_g(j + NB + b, b)
            return carry
        lax.fori_loop(0, n_chunks // NB - 1, grp, 0)
        j_last = n_chunks - NB
        for b in range(NB):
            wait_g(b)
            fire_s(j_last + b, b)
        for b in range(NB):
            wait_s(b)
        plsc.subcore_barrier()

        pltpu.sync_copy(
            acc.at[pl.ds(s * rows_per_tile, rows_per_tile)],
            out_hbm.at[c, pl.ds(s * rows_per_tile, rows_per_tile),
                       pl.ds(0, DP)])

    return pl.kernel(
        body,
        out_type=jax.ShapeDtypeStruct((NC, n_rows, DT), jnp.float32),
        mesh=mesh,
        scratch_types=[
            pltpu.VMEM((n_chunks, CH), jnp.int32),
            pltpu.VMEM((n_chunks, CH), jnp.int32),
            pltpu.VMEM((CH, DP), jnp.float32),
            pltpu.VMEM((CH, DP), jnp.float32),
            pltpu.VMEM((CH, DP), jnp.float32),
            pltpu.VMEM_SHARED((n_rows, DP), jnp.float32),
        ] + [pltpu.SemaphoreType.DMA] * 6,
        compiler_params=pltpu.CompilerParams(use_tc_tiling_on_sc=False),
    )


def _pad_edges(edge_index, n_chunks, dummy_lo, dummy_hi, n_src):
    e_pad = NW * n_chunks * CH
    pad = e_pad - edge_index.shape[1]
    ar = np.arange(pad, dtype=np.int32)
    pad_blk = jnp.asarray(np.stack([
        ar % n_src,
        dummy_lo + ar % (dummy_hi - dummy_lo),
    ]), jnp.int32)
    return jnp.concatenate([edge_index, pad_blk], axis=1).reshape(
        2, NW, n_chunks, CH)


def _tc2_body(p_ref, x_ref, wr_ref, bl_ref, wl_ref, hl_ref, h_ref):
    col = lax.broadcasted_iota(jnp.int32, (1, DT), 1)
    sfull = jnp.where(col < DP, p_ref[0] + p_ref[1], 0.0)
    cnt = jnp.maximum(sfull[:, CNT:CNT + 1], 1.0)
    mean = jnp.where(col < CNT, sfull / cnt, 0.0)
    xw = jnp.dot(x_ref[...], wr_ref[...], preferred_element_type=jnp.float32)
    h = jnp.maximum(mean + bl_ref[...] + xw, 0.0)
    h_ref[...] = h
    hl_ref[...] = (
        jnp.dot(h, wl_ref[...], preferred_element_type=jnp.float32)
        + jnp.where(col % DP == CNT, 1.0, 0.0))


def _tc3_body(q_ref, h_ref, wr_ref, bl_ref, wo_ref, bo_ref, o_ref):
    col = lax.broadcasted_iota(jnp.int32, (1, DT), 1)
    sfull = jnp.where(col < DP, q_ref[0] + q_ref[1], 0.0)
    cnt = jnp.maximum(sfull[:, CNT:CNT + 1], 1.0)
    mean = jnp.where(col < CNT, sfull / cnt, 0.0)
    hw = jnp.dot(h_ref[...], wr_ref[...], preferred_element_type=jnp.float32)
    pre = mean + bl_ref[...] + hw
    out = jnp.dot(pre, wo_ref[...], preferred_element_type=jnp.float32)
    o_ref[...] = jnp.maximum(out + bo_ref[...], 0.0)


def kernel(x, edge_index_0, edge_index_1, edge_attr,
           Wl0, bl0, Wr0, Wl1, bl1, Wr1, W_out, b_out):
    del edge_attr
    f32 = jnp.float32

    def pad64(w):
        out = jnp.zeros((w.shape[0], DP), f32)
        return out.at[:, :w.shape[1]].set(w)

    z64 = jnp.zeros((D_IN, DP), f32)
    wl0_l = jnp.concatenate([pad64(Wl0), z64], axis=1)
    wl0_r = jnp.concatenate([z64, pad64(Wl0)], axis=1)
    wr0_d = jnp.concatenate([pad64(Wr0)] * 2, axis=1)
    wl1_p = jnp.zeros((DP, DP), f32).at[:D_H, :D_H].set(Wl1)
    wr1_p = jnp.zeros((DP, DP), f32).at[:D_H, :D_H].set(Wr1)
    zz = jnp.zeros((DP, DP), f32)
    wl1_bd = jnp.concatenate([
        jnp.concatenate([wl1_p, zz], axis=1),
        jnp.concatenate([zz, wl1_p], axis=1)], axis=0)
    wr1_bd = jnp.concatenate([
        jnp.concatenate([wr1_p, zz], axis=1),
        jnp.concatenate([zz, wr1_p], axis=1)], axis=0)
    wo_s = jnp.zeros((DT, 1), f32).at[:D_H, :].set(W_out)
    bl0_d = jnp.zeros((1, DT), f32).at[0, :D_H].set(bl0)
    bl1_d = jnp.zeros((1, DT), f32).at[0, :D_H].set(bl1)
    bo = b_out.reshape(1, 1)

    e0 = edge_index_0.shape[1]
    e1 = edge_index_1.shape[1]
    nch0 = 3 * _ceil_div(_ceil_div(e0, NW), 3 * CH)
    nch1 = 3 * _ceil_div(_ceil_div(e1, NW), 3 * CH)
    ei0 = _pad_edges(edge_index_0, nch0, N1, R0, N0)
    ei1 = _pad_edges(edge_index_1, nch1, N2, R1, N1)

    xt = _make_table(x, wl0_l, wl0_r, 5000)
    tab0 = xt.reshape(N0, DP)

    p0 = _make_sc_agg(nch0, R0, N0 // 2)(ei0, tab0)

    b2 = 4000
    hl, h = pl.pallas_call(
        _tc2_body,
        grid=(N1 // b2,),
        in_specs=[
            pl.BlockSpec((NC, b2, DT), lambda i: (0, i, 0)),
            pl.BlockSpec((b2, D_IN), lambda i: (i, 0)),
            pl.BlockSpec((D_IN, DT), lambda i: (0, 0)),
            pl.BlockSpec((1, DT), lambda i: (0, 0)),
            pl.BlockSpec((DT, DT), lambda i: (0, 0)),
        ],
        out_specs=[
            pl.BlockSpec((b2, DT), lambda i: (i, 0)),
            pl.BlockSpec((b2, DT), lambda i: (i, 0)),
        ],
        out_shape=[
            jax.ShapeDtypeStruct((N1, DT), f32),
            jax.ShapeDtypeStruct((N1, DT), f32),
        ],
    )(p0, x, wr0_d, bl0_d, wl1_bd)

    tab1 = hl.reshape(2 * N1, DP)
    p1 = _make_sc_agg(nch1, R1, None)(ei1, tab1)

    out = pl.pallas_call(
        _tc3_body,
        grid=(1,),
        in_specs=[
            pl.BlockSpec((NC, N2, DT), lambda i: (0, 0, 0)),
            pl.BlockSpec((N2, DT), lambda i: (0, 0)),
            pl.BlockSpec((DT, DT), lambda i: (0, 0)),
            pl.BlockSpec((1, DT), lambda i: (0, 0)),
            pl.BlockSpec((DT, 1), lambda i: (0, 0)),
            pl.BlockSpec((1, 1), lambda i: (0, 0)),
        ],
        out_specs=pl.BlockSpec((N2, 1), lambda i: (0, 0)),
        out_shape=jax.ShapeDtypeStruct((N2, 1), f32),
    )(p1, h, wr1_bd, bl1_d, wo_s, bo)

    return out


# speedup vs baseline: 1.6981x; 1.0330x over previous
import numpy as np

import jax
import jax.numpy as jnp
from jax import lax
from jax.experimental import pallas as pl
from jax.experimental.pallas import tpu as pltpu
from jax.experimental.pallas import tpu_sc as plsc

N0, N1, N2 = 50000, 20000, 5000
D_IN, D_H = 128, 50
DP = 64
DT = 128
CNT = 50
NC, NS, L = 2, 16, 16
NW = NC * NS
CH = 128

R0 = 20480
R1 = 6144


def _ceil_div(a, b):
    return (a + b - 1) // b


def _tab_body(x1_ref, x2_ref, wl_ref, wr_ref, o_ref):
    acc = (jnp.dot(x1_ref[...], wl_ref[...], preferred_element_type=jnp.float32)
           + jnp.dot(x2_ref[...], wr_ref[...],
                     preferred_element_type=jnp.float32))
    col = lax.broadcasted_iota(jnp.int32, (1, DT), 1)
    o_ref[...] = acc + jnp.where(col % DP == CNT, 1.0, 0.0)


def _make_table(x, w_left, w_right, block_rows):
    n, d = x.shape
    half_blocks = (n // 2) // block_rows
    return pl.pallas_call(
        _tab_body,
        grid=(half_blocks,),
        in_specs=[
            pl.BlockSpec((block_rows, d), lambda i: (i, 0)),
            pl.BlockSpec((block_rows, d),
                         lambda i, hb=half_blocks: (i + hb, 0)),
            pl.BlockSpec((d, DT), lambda i: (0, 0)),
            pl.BlockSpec((d, DT), lambda i: (0, 0)),
        ],
        out_specs=pl.BlockSpec((block_rows, DT), lambda i: (i, 0)),
        out_shape=jax.ShapeDtypeStruct((n // 2, DT), jnp.float32),
    )(x, x, w_left, w_right)


def _make_sc_agg(n_chunks, n_rows, split_half):
    rows_per_tile = n_rows // NS
    n_zch = rows_per_tile // CH
    mesh = plsc.VectorSubcoreMesh(
        core_axis_name="c", subcore_axis_name="s",
        num_cores=NC, num_subcores=NS)
    NB = 3
    assert n_chunks % NB == 0 and n_chunks >= 2 * NB

    def body(edge_hbm, tab_hbm, out_hbm,
             idx_s, idx_d, rows0, rows1, rows2, acc,
             g0, g1, g2, s0, s1, s2):
        rows = (rows0, rows1, rows2)
        gsem = (g0, g1, g2)
        ssem = (s0, s1, s2)
        zbuf = rows0
        c = lax.axis_index("c")
        s = lax.axis_index("s")
        w = c * NS + s

        zv = jnp.zeros((L,), jnp.float32)

        def zrow(i, carry):
            for k in range(DP // L):
                zbuf[i, pl.ds(k * L, L)] = zv
            return carry
        lax.fori_loop(0, CH, zrow, 0)

        def zch(k, carry):
            pltpu.async_copy(
                zbuf, acc.at[pl.ds(s * rows_per_tile + k * CH, CH)], g0)
            return carry
        lax.fori_loop(0, n_zch, zch, 0)
        pltpu.sync_copy(edge_hbm.at[0, w], idx_s)
        pltpu.sync_copy(edge_hbm.at[1, w], idx_d)

        def zdr(k, carry):
            pltpu.make_async_copy(
                zbuf, acc.at[pl.ds(s * rows_per_tile, CH)], g0).wait()
            return carry
        lax.fori_loop(0, n_zch, zdr, 0)
        plsc.subcore_barrier()

        def fire_g(j, b):
            for k in range(CH // L):
                sl = pl.ds(k * L, L)
                v = idx_s[j, sl]
                if split_half is None:
                    idx_s[j, sl] = v * 2
                else:
                    adj = jnp.where(v >= split_half,
                                    jnp.full((L,), 2 * split_half - 1,
                                             jnp.int32),
                                    jnp.zeros((L,), jnp.int32))
                    idx_s[j, sl] = v * 2 - adj
            pltpu.async_copy(tab_hbm.at[idx_s.at[j]], rows[b], gsem[b])

        def wait_g(b):
            pltpu.make_async_copy(
                tab_hbm.at[idx_s.at[0]], rows[b], gsem[b]).wait()

        def fire_s(j, b):
            pltpu.async_copy(
                rows[b], acc.at[idx_d.at[j]], ssem[b], add=True)

        def wait_s(b):
            pltpu.make_async_copy(
                rows[b], acc.at[idx_d.at[0]], ssem[b]).wait()

        for b in range(NB):
            fire_g(b, b)

        def grp(g, carry):
            j = NB * g
            for b in range(NB):
                wait_g(b)
                fire_s(j + b, b)
            for b in range(NB):
                wait_s(b)
                fire_g(j + NB + b, b)
            return carry
        lax.fori_loop(0, n_chunks // NB - 1, grp, 0)
        j_last = n_chunks - NB
        for b in range(NB):
            wait_g(b)
            fire_s(j_last + b, b)
        for b in range(NB):
            wait_s(b)
        plsc.subcore_barrier()

        half = NS // 2
        col_off = (s // half) * DP
        row_base = (s % half) * rows_per_tile
        pltpu.sync_copy(
            acc.at[pl.ds(s * rows_per_tile, rows_per_tile)],
            out_hbm.at[c, pl.ds(row_base, rows_per_tile),
                       pl.ds(col_off, DP)])

    return pl.kernel(
        body,
        out_type=jax.ShapeDtypeStruct((NC, n_rows // 2, DT), jnp.float32),
        mesh=mesh,
        scratch_types=[
            pltpu.VMEM((n_chunks, CH), jnp.int32),
            pltpu.VMEM((n_chunks, CH), jnp.int32),
            pltpu.VMEM((CH, DP), jnp.float32),
            pltpu.VMEM((CH, DP), jnp.float32),
            pltpu.VMEM((CH, DP), jnp.float32),
            pltpu.VMEM_SHARED((n_rows, DP), jnp.float32),
        ] + [pltpu.SemaphoreType.DMA] * 6,
        compiler_params=pltpu.CompilerParams(use_tc_tiling_on_sc=False),
    )


def _pad_edges(edge_index, n_chunks, dummy_lo, dummy_hi, n_src):
    e_pad = NW * n_chunks * CH
    pad = e_pad - edge_index.shape[1]
    ar = np.arange(pad, dtype=np.int32)
    pad_blk = jnp.asarray(np.stack([
        ar % n_src,
        dummy_lo + ar % (dummy_hi - dummy_lo),
    ]), jnp.int32)
    return jnp.concatenate([edge_index, pad_blk], axis=1).reshape(
        2, NW, n_chunks, CH)


def _tc2_body(p_ref, x1_ref, x2_ref, wrl_ref, wrr_ref, bl_ref, wl_ref,
              hl_ref, h_ref):
    col = lax.broadcasted_iota(jnp.int32, (1, DT), 1)
    sfull = p_ref[0] + p_ref[1]
    cnt_l = jnp.maximum(sfull[:, CNT:CNT + 1], 1.0)
    cnt_r = jnp.maximum(sfull[:, DP + CNT:DP + CNT + 1], 1.0)
    cntb = jnp.where(col < DP, cnt_l, cnt_r)
    mean = jnp.where(col % DP < CNT, sfull / cntb, 0.0)
    xw = (jnp.dot(x1_ref[...], wrl_ref[...],
                  preferred_element_type=jnp.float32)
          + jnp.dot(x2_ref[...], wrr_ref[...],
                    preferred_element_type=jnp.float32))
    h = jnp.maximum(mean + bl_ref[...] + xw, 0.0)
    h_ref[...] = h
    hl_ref[...] = (
        jnp.dot(h, wl_ref[...], preferred_element_type=jnp.float32)
        + jnp.where(col % DP == CNT, 1.0, 0.0))


def _tc3_body(q_ref, h1_ref, h2_ref, wr1_ref, wr2_ref, bl_ref, wo_ref,
              bo_ref, o_ref):
    col = lax.broadcasted_iota(jnp.int32, (1, DT), 1)
    sfull = q_ref[0] + q_ref[1]
    cnt_l = jnp.maximum(sfull[:, CNT:CNT + 1], 1.0)
    cnt_r = jnp.maximum(sfull[:, DP + CNT:DP + CNT + 1], 1.0)
    cntb = jnp.where(col < DP, cnt_l, cnt_r)
    mean = jnp.where(col % DP < CNT, sfull / cntb, 0.0)
    hw = (jnp.dot(h1_ref[...], wr1_ref[...],
                  preferred_element_type=jnp.float32)
          + jnp.dot(h2_ref[...], wr2_ref[...],
                    preferred_element_type=jnp.float32))
    pre = mean + bl_ref[...] + hw
    out = jnp.dot(pre, wo_ref[...], preferred_element_type=jnp.float32)
    o_ref[...] = jnp.maximum(out + bo_ref[...], 0.0)


def kernel(x, edge_index_0, edge_index_1, edge_attr,
           Wl0, bl0, Wr0, Wl1, bl1, Wr1, W_out, b_out):
    del edge_attr
    f32 = jnp.float32

    def pad64(w):
        out = jnp.zeros((w.shape[0], DP), f32)
        return out.at[:, :w.shape[1]].set(w)

    z64 = jnp.zeros((D_IN, DP), f32)
    wl0_l = jnp.concatenate([pad64(Wl0), z64], axis=1)
    wl0_r = jnp.concatenate([z64, pad64(Wl0)], axis=1)
    wr0_l = jnp.concatenate([pad64(Wr0), z64], axis=1)
    wr0_r = jnp.concatenate([z64, pad64(Wr0)], axis=1)
    wl1_p = jnp.zeros((DP, DP), f32).at[:D_H, :D_H].set(Wl1)
    wr1_p = jnp.zeros((DP, DP), f32).at[:D_H, :D_H].set(Wr1)
    zz = jnp.zeros((DP, DP), f32)
    wl1_bd = jnp.concatenate([
        jnp.concatenate([wl1_p, zz], axis=1),
        jnp.concatenate([zz, wl1_p], axis=1)], axis=0)
    wr1_w1 = jnp.concatenate([
        jnp.concatenate([wr1_p, zz], axis=1),
        jnp.concatenate([zz, zz], axis=1)], axis=0)
    wr1_w2 = jnp.concatenate([
        jnp.concatenate([zz, wr1_p], axis=1),
        jnp.concatenate([zz, zz], axis=1)], axis=0)
    wo2 = (jnp.zeros((DT, 2), f32)
           .at[:D_H, 0].set(W_out[:, 0])
           .at[DP:DP + D_H, 1].set(W_out[:, 0]))
    bl0_d = jnp.tile(jnp.zeros((1, DP), f32).at[0, :D_H].set(bl0), (1, 2))
    bl1_d = jnp.tile(jnp.zeros((1, DP), f32).at[0, :D_H].set(bl1), (1, 2))
    bo2 = jnp.tile(b_out.reshape(1, 1), (1, 2))

    e0 = edge_index_0.shape[1]
    e1 = edge_index_1.shape[1]
    nch0 = 3 * _ceil_div(_ceil_div(e0, NW), 3 * CH)
    nch1 = 3 * _ceil_div(_ceil_div(e1, NW), 3 * CH)
    ei0 = _pad_edges(edge_index_0, nch0, N1, R0, N0)
    ei1 = _pad_edges(edge_index_1, nch1, N2, R1, N1)

    xt = _make_table(x, wl0_l, wl0_r, 5000)
    tab0 = xt.reshape(N0, DP)

    p0 = _make_sc_agg(nch0, R0, N0 // 2)(ei0, tab0)

    f0 = R0 // 2
    b2 = 2048
    nb2 = f0 // b2
    hl, h = pl.pallas_call(
        _tc2_body,
        grid=(nb2,),
        in_specs=[
            pl.BlockSpec((NC, b2, DT), lambda i: (0, i, 0)),
            pl.BlockSpec((b2, D_IN), lambda i: (i, 0)),
            pl.BlockSpec((b2, D_IN), lambda i, nb=nb2: (i + nb, 0)),
            pl.BlockSpec((D_IN, DT), lambda i: (0, 0)),
            pl.BlockSpec((D_IN, DT), lambda i: (0, 0)),
            pl.BlockSpec((1, DT), lambda i: (0, 0)),
            pl.BlockSpec((DT, DT), lambda i: (0, 0)),
        ],
        out_specs=[
            pl.BlockSpec((b2, DT), lambda i: (i, 0)),
            pl.BlockSpec((b2, DT), lambda i: (i, 0)),
        ],
        out_shape=[
            jax.ShapeDtypeStruct((f0, DT), f32),
            jax.ShapeDtypeStruct((f0, DT), f32),
        ],
    )(p0, x, x, wr0_l, wr0_r, bl0_d, wl1_bd)

    tab1 = hl.reshape(R0, DP)
    p1 = _make_sc_agg(nch1, R1, f0)(ei1, tab1)

    f1 = R1 // 2
    of = pl.pallas_call(
        _tc3_body,
        grid=(1,),
        in_specs=[
            pl.BlockSpec((NC, f1, DT), lambda i: (0, 0, 0)),
            pl.BlockSpec((f1, DT), lambda i: (0, 0)),
            pl.BlockSpec((f1, DT), lambda i: (1, 0)),
            pl.BlockSpec((DT, DT), lambda i: (0, 0)),
            pl.BlockSpec((DT, DT), lambda i: (0, 0)),
            pl.BlockSpec((1, DT), lambda i: (0, 0)),
            pl.BlockSpec((DT, 2), lambda i: (0, 0)),
            pl.BlockSpec((1, 2), lambda i: (0, 0)),
        ],
        out_specs=pl.BlockSpec((f1, 2), lambda i: (0, 0)),
        out_shape=jax.ShapeDtypeStruct((f1, 2), f32),
    )(p1, h, h, wr1_w1, wr1_w2, bl1_d, wo2, bo2)

    return jnp.concatenate([of[:, 0:1], of[:N2 - f1, 1:2]], axis=0)
